# Initial kernel scaffold; baseline (speedup 1.0000x reference)
import jax
import jax.numpy as jnp
from jax.experimental import pallas as pl


def kernel(h, edge_index, W1, b1, W2, b2):
    raise NotImplementedError("write your pallas kernel here")


# speedup vs baseline: 1.1099x; 1.1099x over previous
import functools

import jax
import jax.numpy as jnp
from jax import lax
from jax.experimental import pallas as pl
from jax.experimental.pallas import tpu as pltpu
from jax.experimental.pallas import tpu_sc as plsc


def _proj_body(h_ref, w1at_ref, w1bt_ref, b1_ref, a_ref, b_ref):
    h = h_ref[...]
    a_ref[...] = jnp.dot(h, w1at_ref[...], preferred_element_type=jnp.float32)
    b_ref[...] = (
        jnp.dot(h, w1bt_ref[...], preferred_element_type=jnp.float32)
        + b1_ref[...]
    )


@functools.partial(jax.jit, static_argnames=())
def _project(h, w1at, w1bt, b1r):
    N, D = h.shape
    BN = 2000
    assert N % BN == 0
    return pl.pallas_call(
        _proj_body,
        grid=(N // BN,),
        in_specs=[
            pl.BlockSpec((BN, D), lambda i: (i, 0)),
            pl.BlockSpec((D, D), lambda i: (0, 0)),
            pl.BlockSpec((D, D), lambda i: (0, 0)),
            pl.BlockSpec((1, D), lambda i: (0, 0)),
        ],
        out_specs=[
            pl.BlockSpec((BN, D), lambda i: (i, 0)),
            pl.BlockSpec((BN, D), lambda i: (i, 0)),
        ],
        out_shape=[
            jax.ShapeDtypeStruct((N, D), jnp.float32),
            jax.ShapeDtypeStruct((N, D), jnp.float32),
        ],
    )(h, w1at, w1bt, b1r)


@functools.lru_cache(maxsize=None)
def _make_edge_kernel(N, E, D):
    info = plsc.get_sparse_core_info()
    NC, NS, L = info.num_cores, info.num_subcores, info.num_lanes
    NW = NC * NS
    assert E % NW == 0
    per_w = E // NW
    CH = 80
    assert per_w % CH == 0 and CH % L == 0 and CH % 8 == 0
    n_ch = per_w // CH
    G = CH // L
    DJ = D // L

    mesh = plsc.VectorSubcoreMesh(core_axis_name="c", subcore_axis_name="s")

    @functools.partial(
        pl.kernel,
        mesh=mesh,
        compiler_params=pltpu.CompilerParams(needs_layout_passes=False),
        out_type=[
            jax.ShapeDtypeStruct((E,), jnp.float32),
            jax.ShapeDtypeStruct((E,), jnp.float32),
        ],
        scratch_types=[
            pltpu.VMEM((CH,), jnp.int32),
            pltpu.VMEM((CH,), jnp.int32),
            pltpu.VMEM((CH, D), jnp.float32),
            pltpu.VMEM((CH, D), jnp.float32),
            pltpu.VMEM((CH,), jnp.float32),
            pltpu.VMEM((CH,), jnp.float32),
            pltpu.VMEM((D,), jnp.float32),
            pltpu.VMEM((L,), jnp.float32),
            pltpu.SemaphoreType.DMA,
            pltpu.SemaphoreType.DMA,
        ],
    )
    def edge_kernel(a_hbm, b_hbm, src_hbm, dst_hbm, w2_hbm, b2_hbm,
                    score_hbm, label_hbm,
                    si_v, di_v, a_v, b_v, s_v, l_v, w2_v, b2_v, sem_a, sem_b):
        wid = lax.axis_index("s") * NC + lax.axis_index("c")
        base_w = wid * per_w
        pltpu.sync_copy(w2_hbm, w2_v)
        pltpu.sync_copy(b2_hbm, b2_v)

        def chunk_body(c, carry):
            base = base_w + c * CH
            pltpu.sync_copy(src_hbm.at[pl.ds(base, CH)], si_v)
            pltpu.sync_copy(dst_hbm.at[pl.ds(base, CH)], di_v)
            cpa = pltpu.async_copy(a_hbm.at[si_v], a_v, sem_a)
            cpb = pltpu.async_copy(b_hbm.at[di_v], b_v, sem_b)
            cpa.wait()
            cpb.wait()

            lane = lax.iota(jnp.int32, L)
            b2vec = b2_v[...]

            def group_body(g, gcarry):
                rows = g * L + lane
                acc = b2vec
                for j in range(DJ):
                    wv = w2_v[pl.ds(j * L, L)]
                    for di in range(L):
                        d = j * L + di
                        cols = jnp.full((L,), d, dtype=jnp.int32)
                        av = plsc.load_gather(a_v, [rows, cols])
                        bv = plsc.load_gather(b_v, [rows, cols])
                        t = jnp.maximum(av + bv, 0.0)
                        ti = plsc.bitcast(t, jnp.int32)
                        lsb = jax.lax.shift_right_logical(ti, 16) & 1
                        ri = (ti + (lsb + 0x7FFF)) & jnp.int32(-65536)
                        tr = plsc.bitcast(ri, jnp.float32)
                        acc = acc + tr * wv[di]
                sl = pl.ds(g * L, L)
                s_v[sl] = acc
                l_v[sl] = jnp.where(acc > 0.0, 1.0, 0.0)
                return gcarry

            lax.fori_loop(0, G, group_body, 0, unroll=False)

            pltpu.sync_copy(s_v, score_hbm.at[pl.ds(base, CH)])
            pltpu.sync_copy(l_v, label_hbm.at[pl.ds(base, CH)])
            return carry

        lax.fori_loop(0, n_ch, chunk_body, 0, unroll=False)

    return edge_kernel


def kernel(h, edge_index, W1, b1, W2, b2):
    N, D = h.shape
    E = edge_index.shape[1]
    w1at = W1[:, :D].T.astype(jnp.bfloat16)
    w1bt = W1[:, D:].T.astype(jnp.bfloat16)
    b1r = b1.reshape(1, D)
    A, B = _project(h.astype(jnp.bfloat16), w1at, w1bt, b1r)
    w2f = W2.reshape(D)
    w2i = jax.lax.bitcast_convert_type(w2f, jnp.int32)
    w2lsb = jax.lax.shift_right_logical(w2i, 16) & 1
    w2r = (w2i + (w2lsb + 0x7FFF)) & jnp.int32(-65536)
    w2 = jax.lax.bitcast_convert_type(w2r, jnp.float32)
    b2bc = jnp.broadcast_to(b2, (16,)).astype(jnp.float32)
    src = edge_index[0]
    dst = edge_index[1]
    score, label = _make_edge_kernel(N, E, D)(A, B, src, dst, w2, b2bc)
    return score, label


# speedup vs baseline: 1.3523x; 1.2185x over previous
import functools

import jax
import jax.numpy as jnp
from jax import lax
from jax.experimental import pallas as pl
from jax.experimental.pallas import tpu as pltpu
from jax.experimental.pallas import tpu_sc as plsc


def _proj_body(h_ref, w1at_ref, w1bt_ref, b1_ref, a_ref, b_ref):
    h = h_ref[...]
    a_ref[...] = jnp.dot(h, w1at_ref[...], preferred_element_type=jnp.float32)
    b_ref[...] = (
        jnp.dot(h, w1bt_ref[...], preferred_element_type=jnp.float32)
        + b1_ref[...]
    )


@functools.partial(jax.jit, static_argnames=())
def _project(h, w1at, w1bt, b1r):
    N, D = h.shape
    BN = 2000
    assert N % BN == 0
    return pl.pallas_call(
        _proj_body,
        grid=(N // BN,),
        in_specs=[
            pl.BlockSpec((BN, D), lambda i: (i, 0)),
            pl.BlockSpec((D, D), lambda i: (0, 0)),
            pl.BlockSpec((D, D), lambda i: (0, 0)),
            pl.BlockSpec((1, D), lambda i: (0, 0)),
        ],
        out_specs=[
            pl.BlockSpec((BN, D), lambda i: (i, 0)),
            pl.BlockSpec((BN, D), lambda i: (i, 0)),
        ],
        out_shape=[
            jax.ShapeDtypeStruct((N, D), jnp.float32),
            jax.ShapeDtypeStruct((N, D), jnp.float32),
        ],
    )(h, w1at, w1bt, b1r)


@functools.lru_cache(maxsize=None)
def _make_edge_kernel(N, E, D):
    info = plsc.get_sparse_core_info()
    NC, NS, L = info.num_cores, info.num_subcores, info.num_lanes
    NW = NC * NS
    assert E % NW == 0
    per_w = E // NW
    CH = 80
    assert per_w % CH == 0 and CH % L == 0 and CH % 8 == 0
    n_ch = per_w // CH
    G = CH // L
    DJ = D // L

    mesh = plsc.VectorSubcoreMesh(core_axis_name="c", subcore_axis_name="s")

    @functools.partial(
        pl.kernel,
        mesh=mesh,
        compiler_params=pltpu.CompilerParams(needs_layout_passes=False),
        out_type=[
            jax.ShapeDtypeStruct((E,), jnp.float32),
            jax.ShapeDtypeStruct((E,), jnp.float32),
        ],
        scratch_types=[
            pltpu.VMEM((per_w,), jnp.int32),
            pltpu.VMEM((per_w,), jnp.int32),
            pltpu.VMEM((CH, D), jnp.float32),
            pltpu.VMEM((CH, D), jnp.float32),
            pltpu.VMEM((CH, D), jnp.float32),
            pltpu.VMEM((CH, D), jnp.float32),
            pltpu.VMEM((per_w,), jnp.float32),
            pltpu.VMEM((per_w,), jnp.float32),
            pltpu.VMEM((D,), jnp.float32),
            pltpu.VMEM((L,), jnp.float32),
            pltpu.SemaphoreType.DMA,
            pltpu.SemaphoreType.DMA,
            pltpu.SemaphoreType.DMA,
            pltpu.SemaphoreType.DMA,
        ],
    )
    def edge_kernel(a_hbm, b_hbm, src_hbm, dst_hbm, w2_hbm, b2_hbm,
                    score_hbm, label_hbm,
                    si_v, di_v, a0, b0, a1, b1, s_v, l_v, w2_v, b2_v,
                    sa0, sb0, sa1, sb1):
        wid = lax.axis_index("s") * NC + lax.axis_index("c")
        base_w = wid * per_w
        pltpu.sync_copy(w2_hbm, w2_v)
        pltpu.sync_copy(b2_hbm, b2_v)
        pltpu.sync_copy(src_hbm.at[pl.ds(base_w, per_w)], si_v)
        pltpu.sync_copy(dst_hbm.at[pl.ds(base_w, per_w)], di_v)

        lane = lax.iota(jnp.int32, L)
        b2vec = b2_v[...]

        def issue(c, ab, bb, sa, sb):
            off = pl.multiple_of(c * CH, CH)
            pltpu.async_copy(a_hbm.at[si_v.at[pl.ds(off, CH)]], ab, sa)
            pltpu.async_copy(b_hbm.at[di_v.at[pl.ds(off, CH)]], bb, sb)

        def wait(ab, bb, sa, sb):
            pltpu.make_async_copy(a_hbm.at[pl.ds(0, CH)], ab, sa).wait()
            pltpu.make_async_copy(b_hbm.at[pl.ds(0, CH)], bb, sb).wait()

        def compute(c, ab, bb):
            def group_body(g, gcarry):
                rows = g * L + lane
                acc = b2vec
                for j in range(DJ):
                    wv = w2_v[pl.ds(j * L, L)]
                    for di in range(L):
                        d = j * L + di
                        cols = jnp.full((L,), d, dtype=jnp.int32)
                        av = plsc.load_gather(ab, [rows, cols])
                        bv = plsc.load_gather(bb, [rows, cols])
                        t = jnp.maximum(av + bv, 0.0)
                        ti = plsc.bitcast(t, jnp.int32)
                        lsb = jax.lax.shift_right_logical(ti, 16) & 1
                        ri = (ti + (lsb + 0x7FFF)) & jnp.int32(-65536)
                        tr = plsc.bitcast(ri, jnp.float32)
                        acc = acc + tr * wv[di]
                sl = pl.ds(pl.multiple_of(c * CH + g * L, L), L)
                s_v[sl] = acc
                l_v[sl] = jnp.where(acc > 0.0, 1.0, 0.0)
                return gcarry

            lax.fori_loop(0, G, group_body, 0, unroll=False)

        issue(0, a0, b0, sa0, sb0)

        def pair_body(i, carry):
            c = 2 * i
            issue(c + 1, a1, b1, sa1, sb1)
            wait(a0, b0, sa0, sb0)
            compute(c, a0, b0)
            issue(c + 2, a0, b0, sa0, sb0)
            wait(a1, b1, sa1, sb1)
            compute(c + 1, a1, b1)
            return carry

        lax.fori_loop(0, (n_ch - 1) // 2, pair_body, 0, unroll=False)
        wait(a0, b0, sa0, sb0)
        compute(n_ch - 1, a0, b0)

        pltpu.sync_copy(s_v, score_hbm.at[pl.ds(base_w, per_w)])
        pltpu.sync_copy(l_v, label_hbm.at[pl.ds(base_w, per_w)])

    return edge_kernel


def kernel(h, edge_index, W1, b1, W2, b2):
    N, D = h.shape
    E = edge_index.shape[1]
    w1at = W1[:, :D].T.astype(jnp.bfloat16)
    w1bt = W1[:, D:].T.astype(jnp.bfloat16)
    b1r = b1.reshape(1, D)
    A, B = _project(h.astype(jnp.bfloat16), w1at, w1bt, b1r)
    w2f = W2.reshape(D)
    w2i = jax.lax.bitcast_convert_type(w2f, jnp.int32)
    w2lsb = jax.lax.shift_right_logical(w2i, 16) & 1
    w2r = (w2i + (w2lsb + 0x7FFF)) & jnp.int32(-65536)
    w2 = jax.lax.bitcast_convert_type(w2r, jnp.float32)
    b2bc = jnp.broadcast_to(b2, (16,)).astype(jnp.float32)
    src = edge_index[0]
    dst = edge_index[1]
    score, label = _make_edge_kernel(N, E, D)(A, B, src, dst, w2, b2bc)
    return score, label


# speedup vs baseline: 8.5282x; 6.3063x over previous
import functools

import jax
import jax.numpy as jnp
from jax import lax
from jax.experimental import pallas as pl
from jax.experimental.pallas import tpu as pltpu
from jax.experimental.pallas import tpu_sc as plsc


def _proj_body(h_ref, w1_ref, b1_ref, a_ref, b_ref):
    D = h_ref.shape[1]
    h = h_ref[...].astype(jnp.bfloat16)
    w1 = w1_ref[...].astype(jnp.bfloat16)
    dn = (((1,), (1,)), ((), ()))
    a_ref[...] = jax.lax.dot_general(
        h, w1[:, :D], dn, preferred_element_type=jnp.float32)
    b_ref[...] = (
        jax.lax.dot_general(
            h, w1[:, D:], dn, preferred_element_type=jnp.float32)
        + b1_ref[...]
    )


@functools.partial(jax.jit, static_argnames=())
def _project(h, w1, b1r):
    N, D = h.shape
    BN = 2000
    assert N % BN == 0
    return pl.pallas_call(
        _proj_body,
        grid=(N // BN,),
        in_specs=[
            pl.BlockSpec((BN, D), lambda i: (i, 0)),
            pl.BlockSpec((D, 2 * D), lambda i: (0, 0)),
            pl.BlockSpec((1, D), lambda i: (0, 0)),
        ],
        out_specs=[
            pl.BlockSpec((BN, D), lambda i: (i, 0)),
            pl.BlockSpec((BN, D), lambda i: (i, 0)),
        ],
        out_shape=[
            jax.ShapeDtypeStruct((N, D), jnp.float32),
            jax.ShapeDtypeStruct((N, D), jnp.float32),
        ],
    )(h, w1, b1r)


@functools.lru_cache(maxsize=None)
def _make_edge_kernel(N, E, D):
    info = plsc.get_sparse_core_info()
    NC, NS, L = info.num_cores, info.num_subcores, info.num_lanes
    NW = NC * NS
    assert E % NW == 0
    per_w = E // NW
    CH = 80
    assert per_w % CH == 0 and CH % L == 0 and CH % 8 == 0
    n_ch = per_w // CH
    assert n_ch % 2 == 1
    G = CH // L
    DJ = D // L

    mesh = plsc.VectorSubcoreMesh(core_axis_name="c", subcore_axis_name="s")

    @functools.partial(
        pl.kernel,
        mesh=mesh,
        compiler_params=pltpu.CompilerParams(needs_layout_passes=False),
        out_type=[
            jax.ShapeDtypeStruct((E,), jnp.float32),
            jax.ShapeDtypeStruct((E,), jnp.float32),
        ],
        scratch_types=[
            pltpu.VMEM((per_w,), jnp.int32),
            pltpu.VMEM((per_w,), jnp.int32),
            pltpu.VMEM((CH, D), jnp.float32),
            pltpu.VMEM((CH, D), jnp.float32),
            pltpu.VMEM((CH, D), jnp.float32),
            pltpu.VMEM((CH, D), jnp.float32),
            pltpu.VMEM((per_w,), jnp.float32),
            pltpu.VMEM((per_w,), jnp.float32),
            pltpu.VMEM((D,), jnp.float32),
            pltpu.VMEM((L,), jnp.float32),
            pltpu.VMEM((CH * L,), jnp.float32),
            pltpu.SemaphoreType.DMA,
            pltpu.SemaphoreType.DMA,
            pltpu.SemaphoreType.DMA,
            pltpu.SemaphoreType.DMA,
        ],
    )
    def edge_kernel(a_hbm, b_hbm, src_hbm, dst_hbm, w2_hbm, b2_hbm,
                    score_hbm, label_hbm,
                    si_v, di_v, a0, b0, a1, b1, s_v, l_v, w2_v, b2_v,
                    p_scr, sa0, sb0, sa1, sb1):
        wid = lax.axis_index("s") * NC + lax.axis_index("c")
        base_w = wid * per_w

        pltpu.sync_copy(w2_hbm, w2_v)
        pltpu.sync_copy(b2_hbm, b2_v)
        pltpu.sync_copy(src_hbm.at[pl.ds(base_w, per_w)], si_v)
        pltpu.sync_copy(dst_hbm.at[pl.ds(base_w, per_w)], di_v)

        lane = lax.iota(jnp.int32, L)
        b2vec = b2_v[...]

        for j in range(DJ):
            wsl = pl.ds(j * L, L)
            wi = plsc.bitcast(w2_v[wsl], jnp.int32)
            wlsb = jax.lax.shift_right_logical(wi, 16) & 1
            wr = (wi + (wlsb + 0x7FFF)) & jnp.int32(-65536)
            w2_v[wsl] = plsc.bitcast(wr, jnp.float32)

        def issue(c, ab, bb, sa, sb):
            off = pl.multiple_of(c * CH, CH)
            pltpu.async_copy(a_hbm.at[si_v.at[pl.ds(off, CH)]], ab, sa)
            pltpu.async_copy(b_hbm.at[di_v.at[pl.ds(off, CH)]], bb, sb)

        def wait(ab, bb, sa, sb):
            pltpu.make_async_copy(a_hbm.at[pl.ds(0, CH)], ab, sa).wait()
            pltpu.make_async_copy(b_hbm.at[pl.ds(0, CH)], bb, sb).wait()

        def compute(c, ab, bb):
            w2vs = [w2_v[pl.ds(j * L, L)] for j in range(DJ)]
            lane16 = lane * L
            zero = jnp.zeros((L,), jnp.float32)

            @plsc.parallel_loop(0, G)
            def group_body(g):
                e0 = g * L
                partials = []
                for k in range(L):
                    e = e0 + k
                    parts = []
                    for j in range(DJ):
                        av = ab[e, pl.ds(j * L, L)]
                        bv = bb[e, pl.ds(j * L, L)]
                        t = jnp.maximum(av + bv, 0.0)
                        ti = plsc.bitcast(t, jnp.int32)
                        ri = (ti + 0x8000) & jnp.int32(-65536)
                        tr = plsc.bitcast(ri, jnp.float32)
                        parts.append(tr * w2vs[j])
                    s01 = (parts[0] + parts[1]) + (parts[2] + parts[3])
                    s23 = (parts[4] + parts[5]) + (parts[6] + parts[7])
                    partials.append(s01 + s23)

                goff = pl.multiple_of(g * (L * L), L)
                for k in range(L):
                    p_scr[pl.ds(goff + k * L, L)] = partials[k]
                lanebase = lane16 + goff
                accs = [b2vec] + [zero] * 7
                for l in range(L):
                    cv = plsc.load_gather(p_scr, [lanebase + l])
                    accs[l & 7] = accs[l & 7] + cv
                acc = (((accs[0] + accs[1]) + (accs[2] + accs[3]))
                       + ((accs[4] + accs[5]) + (accs[6] + accs[7])))
                sl = pl.ds(pl.multiple_of(c * CH + g * L, L), L)
                s_v[sl] = acc
                l_v[sl] = jnp.where(acc > 0.0, 1.0, 0.0)

        issue(0, a0, b0, sa0, sb0)

        def pair_body(i, carry):
            c = 2 * i
            issue(c + 1, a1, b1, sa1, sb1)
            wait(a0, b0, sa0, sb0)
            compute(c, a0, b0)
            issue(c + 2, a0, b0, sa0, sb0)
            wait(a1, b1, sa1, sb1)
            compute(c + 1, a1, b1)
            return carry

        lax.fori_loop(0, (n_ch - 1) // 2, pair_body, 0, unroll=False)
        wait(a0, b0, sa0, sb0)
        compute(n_ch - 1, a0, b0)

        pltpu.sync_copy(s_v, score_hbm.at[pl.ds(base_w, per_w)])
        pltpu.sync_copy(l_v, label_hbm.at[pl.ds(base_w, per_w)])

    return edge_kernel


def kernel(h, edge_index, W1, b1, W2, b2):
    N, D = h.shape
    E = edge_index.shape[1]
    b1r = b1.reshape(1, D)
    A, B = _project(h, W1, b1r)
    w2 = W2.reshape(D)
    b2bc = jnp.broadcast_to(b2, (16,)).astype(jnp.float32)
    src = edge_index[0]
    dst = edge_index[1]
    score, label = _make_edge_kernel(N, E, D)(A, B, src, dst, w2, b2bc)
    return score, label
